# Initial kernel scaffold; baseline (speedup 1.0000x reference)
import jax
import jax.numpy as jnp
from jax.experimental import pallas as pl


def kernel(x, edge_index, W1, b1, W2, b2):
    raise NotImplementedError("write your pallas kernel here")


# speedup vs baseline: 20.6833x; 20.6833x over previous
import functools

import jax
import jax.numpy as jnp
from jax import lax
from jax.experimental import pallas as pl
from jax.experimental.pallas import tpu as pltpu
from jax.experimental.pallas import tpu_sc as plsc

N = 100000
E = 1600000
D_IN = 32
D_HID = 16
D_OUT = 8

NC = 2
NS = 16
NPAD = 100352
EPAD = 1605632
ER = EPAD // 128
ROWS_W = ER // (NC * NS)
KIN = 8
GOUT = ROWS_W // KIN
TPS = NPAD // NS
NZC = 14
ZCH = TPS // NZC

@functools.cache
def _mesh():
    return plsc.VectorSubcoreMesh(
        core_axis_name="c", subcore_axis_name="s", num_cores=NC, num_subcores=NS
    )


def _worker_base(c, s):
    return (c * NS + s) * ROWS_W


@functools.cache
def _make_deg_kernel():
    return functools.partial(
        pl.kernel,
        out_type=jax.ShapeDtypeStruct((NC * 2 * NPAD,), jnp.float32),
        mesh=_mesh(),
        compiler_params=pltpu.CompilerParams(use_tc_tiling_on_sc=False),
        scratch_types=[
        pltpu.VMEM((KIN, 128), jnp.int32),
        pltpu.VMEM((KIN, 128), jnp.int32),
        pltpu.VMEM((128,), jnp.float32),
        pltpu.VMEM((ZCH,), jnp.float32),
            pltpu.VMEM_SHARED((NPAD,), jnp.float32),
            pltpu.VMEM_SHARED((NPAD,), jnp.float32),
        ],
    )(_deg_body)


def _deg_body(srcr, dstr, zer, one, out, src_v, dst_v, ones_v, zb, dego, degi):
    c = lax.axis_index("c")
    s = lax.axis_index("s")
    pltpu.sync_copy(zer, zb)
    pltpu.sync_copy(one, ones_v)
    for i in range(NZC):
        off = s * TPS + i * ZCH
        pltpu.sync_copy(zb, dego.at[pl.ds(off, ZCH)])
        pltpu.sync_copy(zb, degi.at[pl.ds(off, ZCH)])
    plsc.subcore_barrier()
    w = _worker_base(c, s)

    def body(g, carry):
        base = w + g * KIN
        pltpu.sync_copy(srcr.at[pl.ds(base, KIN)], src_v)
        pltpu.sync_copy(dstr.at[pl.ds(base, KIN)], dst_v)
        for j in range(KIN):
            pltpu.sync_copy(ones_v, dego.at[src_v.at[j]], add=True)
            pltpu.sync_copy(ones_v, degi.at[dst_v.at[j]], add=True)
        return carry

    lax.fori_loop(0, GOUT, body, 0)
    plsc.subcore_barrier()
    for i in range(NZC):
        off = s * TPS + i * ZCH
        pltpu.sync_copy(
            dego.at[pl.ds(off, ZCH)], out.at[pl.ds(c * 2 * NPAD + off, ZCH)]
        )
        pltpu.sync_copy(
            degi.at[pl.ds(off, ZCH)], out.at[pl.ds((c * 2 + 1) * NPAD + off, ZCH)]
        )


@functools.cache
def _make_edge_kernel(d):
    @functools.partial(
        pl.kernel,
        out_type=jax.ShapeDtypeStruct((NC, NPAD, d), jnp.float32),
        mesh=_mesh(),
        compiler_params=pltpu.CompilerParams(use_tc_tiling_on_sc=False),
        scratch_types=[
            pltpu.VMEM((KIN, 128), jnp.int32),
            pltpu.VMEM((KIN, 128), jnp.int32),
            pltpu.VMEM((KIN * 128, d), jnp.float32),
            pltpu.VMEM((ZCH, d), jnp.float32),
            pltpu.VMEM_SHARED((NPAD, d), jnp.float32),
            pltpu.SemaphoreType.DMA,
        ],
    )
    def _edge_kernel(srcr, dstr, q, zer, out, src_v, dst_v, rows_v, zb, agg, sem):
        c = lax.axis_index("c")
        s = lax.axis_index("s")
        pltpu.sync_copy(zer, zb)
        for i in range(NZC):
            off = s * TPS + i * ZCH
            pltpu.sync_copy(zb, agg.at[pl.ds(off, ZCH)])
        plsc.subcore_barrier()
        w = _worker_base(c, s)

        def body(g, carry):
            base = w + g * KIN
            pltpu.sync_copy(srcr.at[pl.ds(base, KIN)], src_v)
            pltpu.sync_copy(dstr.at[pl.ds(base, KIN)], dst_v)
            descs = []
            for j in range(KIN):
                descs.append(
                    pltpu.async_copy(
                        q.at[src_v.at[j]], rows_v.at[pl.ds(j * 128, 128)], sem
                    )
                )
            for dsc in descs:
                dsc.wait()
            for j in range(KIN):
                pltpu.sync_copy(
                    rows_v.at[pl.ds(j * 128, 128)], agg.at[dst_v.at[j]], add=True
                )
            return carry

        lax.fori_loop(0, GOUT, body, 0)
        plsc.subcore_barrier()
        for i in range(NZC):
            off = s * TPS + i * ZCH
            pltpu.sync_copy(agg.at[pl.ds(off, ZCH)], out.at[c, pl.ds(off, ZCH)])

    return _edge_kernel

R = 2048


def _q1_body(x_ref, deg_ref, w_ref, o_ref):
    deg = deg_ref[...]
    n_out = lax.rsqrt(jnp.maximum(deg[0, 0] + deg[1, 0], 1.0))
    q = jnp.dot(x_ref[...], w_ref[...], preferred_element_type=jnp.float32)
    o_ref[...] = q * n_out


def _mid_body(agg_ref, deg_ref, b_ref, w_ref, o_ref):
    deg = deg_ref[...]
    n_in = lax.rsqrt(jnp.maximum(deg[0, 1] + deg[1, 1], 1.0))
    n_out = lax.rsqrt(jnp.maximum(deg[0, 0] + deg[1, 0], 1.0))
    agg = agg_ref[...]
    h = jax.nn.relu((agg[0] + agg[1]) * n_in + b_ref[...])
    q = jnp.dot(h, w_ref[...], preferred_element_type=jnp.float32)
    o_ref[...] = q * n_out


def _out_body(agg_ref, deg_ref, b_ref, o_ref):
    deg = deg_ref[...]
    n_in = lax.rsqrt(jnp.maximum(deg[0, 1] + deg[1, 1], 1.0))
    agg = agg_ref[...]
    o_ref[...] = (agg[0] + agg[1]) * n_in + b_ref[...]


def _tc_q1(xp, degp, w1):
    grid = NPAD // R
    return pl.pallas_call(
        _q1_body,
        grid=(grid,),
        in_specs=[
            pl.BlockSpec((R, D_IN), lambda i: (i, 0)),
            pl.BlockSpec((NC, 2, R, 1), lambda i: (0, 0, i, 0)),
            pl.BlockSpec((D_IN, D_HID), lambda i: (0, 0)),
        ],
        out_specs=pl.BlockSpec((R, D_HID), lambda i: (i, 0)),
        out_shape=jax.ShapeDtypeStruct((NPAD, D_HID), jnp.float32),
    )(xp, degp, w1)


def _tc_mid(aggp, degp, b1, w2):
    grid = NPAD // R
    return pl.pallas_call(
        _mid_body,
        grid=(grid,),
        in_specs=[
            pl.BlockSpec((NC, R, D_HID), lambda i: (0, i, 0)),
            pl.BlockSpec((NC, 2, R, 1), lambda i: (0, 0, i, 0)),
            pl.BlockSpec((1, D_HID), lambda i: (0, 0)),
            pl.BlockSpec((D_HID, D_OUT), lambda i: (0, 0)),
        ],
        out_specs=pl.BlockSpec((R, D_OUT), lambda i: (i, 0)),
        out_shape=jax.ShapeDtypeStruct((NPAD, D_OUT), jnp.float32),
    )(aggp, degp, b1, w2)


def _tc_out(aggp, degp, b2):
    grid = NPAD // R
    return pl.pallas_call(
        _out_body,
        grid=(grid,),
        in_specs=[
            pl.BlockSpec((NC, R, D_OUT), lambda i: (0, i, 0)),
            pl.BlockSpec((NC, 2, R, 1), lambda i: (0, 0, i, 0)),
            pl.BlockSpec((1, D_OUT), lambda i: (0, 0)),
        ],
        out_specs=pl.BlockSpec((R, D_OUT), lambda i: (i, 0)),
        out_shape=jax.ShapeDtypeStruct((NPAD, D_OUT), jnp.float32),
    )(aggp, degp, b2)


@jax.jit
def kernel(x, edge_index, W1, b1, W2, b2):
    pad = jnp.full((EPAD - E,), N, dtype=jnp.int32)
    srcr = jnp.concatenate([edge_index[0], pad]).reshape(ER, 128)
    dstr = jnp.concatenate([edge_index[1], pad]).reshape(ER, 128)
    xp = jnp.pad(x, ((0, NPAD - N), (0, 0)))

    degp = _make_deg_kernel()(
        srcr, dstr, jnp.zeros((ZCH,), jnp.float32), jnp.ones((128,), jnp.float32)
    )
    degp4 = degp.reshape(NC, 2, NPAD, 1)
    del degp

    q1 = _tc_q1(xp, degp4, W1)
    aggp1 = _make_edge_kernel(D_HID)(
        srcr, dstr, q1, jnp.zeros((ZCH, D_HID), jnp.float32)
    )
    q2 = _tc_mid(aggp1, degp4, b1.reshape(1, D_HID), W2)
    aggp2 = _make_edge_kernel(D_OUT)(
        srcr, dstr, q2, jnp.zeros((ZCH, D_OUT), jnp.float32)
    )
    out = _tc_out(aggp2, degp4, b2.reshape(1, D_OUT))
    return out[:N]


# speedup vs baseline: 44.0706x; 2.1307x over previous
import functools

import jax
import jax.numpy as jnp
from jax import lax
from jax.experimental import pallas as pl
from jax.experimental.pallas import tpu as pltpu
from jax.experimental.pallas import tpu_sc as plsc

N = 100000
E = 1600000
D_IN = 32
D_HID = 16
D_OUT = 8

NC = 2
NS = 16
NPAD = 100352
EPAD = 1605632
ER = EPAD // 128
ROWS_W = ER // (NC * NS)
KIN = 8
GOUT = ROWS_W // KIN
TPS = NPAD // NS
NZC = 14
ZCH = TPS // NZC

@functools.cache
def _mesh():
    return plsc.VectorSubcoreMesh(
        core_axis_name="c", subcore_axis_name="s", num_cores=NC, num_subcores=NS
    )


def _worker_base(c, s):
    return (c * NS + s) * ROWS_W


@functools.cache
def _make_deg_kernel():
    return functools.partial(
        pl.kernel,
        out_type=jax.ShapeDtypeStruct((NC * 2 * NPAD,), jnp.float32),
        mesh=_mesh(),
        compiler_params=pltpu.CompilerParams(use_tc_tiling_on_sc=False),
        scratch_types=[
            pltpu.VMEM((2, KIN, 128), jnp.int32),
            pltpu.VMEM((2, KIN, 128), jnp.int32),
            pltpu.VMEM((128,), jnp.float32),
            pltpu.VMEM((ZCH,), jnp.float32),
            pltpu.VMEM_SHARED((NPAD,), jnp.float32),
            pltpu.VMEM_SHARED((NPAD,), jnp.float32),
            pltpu.SemaphoreType.DMA,
            pltpu.SemaphoreType.DMA,
        ],
    )(_deg_body)


def _deg_body(srcr, dstr, zer, one, out, src_v, dst_v, ones_v, zb, dego, degi, ss0, ss1):
    ssems = (ss0, ss1)
    c = lax.axis_index("c")
    s = lax.axis_index("s")
    pltpu.sync_copy(zer, zb)
    pltpu.sync_copy(one, ones_v)
    for i in range(NZC):
        off = s * TPS + i * ZCH
        pltpu.sync_copy(zb, dego.at[pl.ds(off, ZCH)])
        pltpu.sync_copy(zb, degi.at[pl.ds(off, ZCH)])
    plsc.subcore_barrier()
    w = _worker_base(c, s)

    def stage(g, p):
        base = w + g * KIN
        pltpu.sync_copy(srcr.at[pl.ds(base, KIN)], src_v.at[p])
        pltpu.sync_copy(dstr.at[pl.ds(base, KIN)], dst_v.at[p])

    def fire(p):
        for j in range(KIN):
            pltpu.async_copy(ones_v, dego.at[src_v.at[p, j]], ssems[p], add=True)
            pltpu.async_copy(ones_v, degi.at[dst_v.at[p, j]], ssems[p], add=True)

    def drain(p):
        for j in range(KIN):
            pltpu.make_async_copy(ones_v, dego.at[src_v.at[p, j]], ssems[p]).wait()
            pltpu.make_async_copy(ones_v, degi.at[dst_v.at[p, j]], ssems[p]).wait()

    stage(0, 0)
    stage(1, 1)
    fire(0)

    def body(i, carry):
        g = i * 2

        @pl.when(g + 1 < GOUT)
        def _():
            fire(1)

        drain(0)

        @pl.when(g + 2 < GOUT)
        def _():
            stage(g + 2, 0)
            fire(0)

        @pl.when(g + 1 < GOUT)
        def _():
            drain(1)

        @pl.when(g + 3 < GOUT)
        def _():
            stage(g + 3, 1)

        return carry

    lax.fori_loop(0, (GOUT + 1) // 2, body, 0)
    plsc.subcore_barrier()
    for i in range(NZC):
        off = s * TPS + i * ZCH
        pltpu.sync_copy(
            dego.at[pl.ds(off, ZCH)], out.at[pl.ds(c * 2 * NPAD + off, ZCH)]
        )
        pltpu.sync_copy(
            degi.at[pl.ds(off, ZCH)], out.at[pl.ds((c * 2 + 1) * NPAD + off, ZCH)]
        )


@functools.cache
def _make_edge_kernel(d):
    kin = 4 if d == 16 else 8
    gout = ROWS_W // kin

    @functools.partial(
        pl.kernel,
        out_type=jax.ShapeDtypeStruct((NC, NPAD, d), jnp.float32),
        mesh=_mesh(),
        compiler_params=pltpu.CompilerParams(use_tc_tiling_on_sc=False),
        scratch_types=[
            pltpu.VMEM((2, kin, 128), jnp.int32),
            pltpu.VMEM((2, kin, 128), jnp.int32),
            pltpu.VMEM((2, kin * 128, d), jnp.float32),
            pltpu.VMEM_SHARED((NPAD, d), jnp.float32),
            pltpu.SemaphoreType.DMA,
            pltpu.SemaphoreType.DMA,
            pltpu.SemaphoreType.DMA,
            pltpu.SemaphoreType.DMA,
        ],
    )
    def _edge_kernel(
        srcr, dstr, q, zer, out, src_v, dst_v, rows_v, agg, gs0, gs1, ss0, ss1
    ):
        gsems = (gs0, gs1)
        ssems = (ss0, ss1)
        c = lax.axis_index("c")
        s = lax.axis_index("s")
        pltpu.sync_copy(zer, rows_v.at[0, pl.ds(0, ZCH)])
        for i in range(NZC):
            off = s * TPS + i * ZCH
            pltpu.sync_copy(rows_v.at[0, pl.ds(0, ZCH)], agg.at[pl.ds(off, ZCH)])
        plsc.subcore_barrier()
        w = _worker_base(c, s)

        def stage(g, p):
            base = w + g * kin
            pltpu.sync_copy(srcr.at[pl.ds(base, kin)], src_v.at[p])
            pltpu.sync_copy(dstr.at[pl.ds(base, kin)], dst_v.at[p])

        def fire_gathers(p):
            for j in range(kin):
                pltpu.async_copy(
                    q.at[src_v.at[p, j]],
                    rows_v.at[p, pl.ds(j * 128, 128)],
                    gsems[p],
                )

        def drain_gathers(p):
            for j in range(kin):
                pltpu.make_async_copy(
                    q.at[src_v.at[p, j]],
                    rows_v.at[p, pl.ds(j * 128, 128)],
                    gsems[p],
                ).wait()

        def fire_scatters(p):
            for j in range(kin):
                pltpu.async_copy(
                    rows_v.at[p, pl.ds(j * 128, 128)],
                    agg.at[dst_v.at[p, j]],
                    ssems[p],
                    add=True,
                )

        def drain_scatters(p):
            for j in range(kin):
                pltpu.make_async_copy(
                    rows_v.at[p, pl.ds(j * 128, 128)],
                    agg.at[dst_v.at[p, j]],
                    ssems[p],
                ).wait()

        stage(0, 0)
        stage(1, 1)
        fire_gathers(0)

        def half(g, p):
            @pl.when(g + 1 < gout)
            def _():
                fire_gathers(1 - p)

            drain_gathers(p)
            fire_scatters(p)

            @pl.when(g + 2 < gout)
            def _():
                stage(g + 2, p)

        def body(i, carry):
            g = i * 2

            @pl.when(g > 0)
            def _():
                drain_scatters(1)

            half(g, 0)
            drain_scatters(0)
            half(g + 1, 1)
            return carry

        lax.fori_loop(0, gout // 2, body, 0)
        if gout % 2 == 1:
            drain_scatters(1)
            drain_gathers(0)
            fire_scatters(0)
            drain_scatters(0)
        else:
            drain_scatters(1)
        plsc.subcore_barrier()
        for i in range(NZC):
            off = s * TPS + i * ZCH
            pltpu.sync_copy(agg.at[pl.ds(off, ZCH)], out.at[c, pl.ds(off, ZCH)])

    return _edge_kernel

NR = NPAD // 128
N8 = NPAD // 8
N16 = NPAD // 16


def _norm_body(deg_ref, no_ref, ni_ref):
    deg = deg_ref[...]
    no_ref[...] = lax.rsqrt(jnp.maximum(deg[0, 0] + deg[1, 0], 1.0))
    ni_ref[...] = lax.rsqrt(jnp.maximum(deg[0, 1] + deg[1, 1], 1.0))


def _tc_norms(degp4):
    return pl.pallas_call(
        _norm_body,
        out_shape=[
            jax.ShapeDtypeStruct((NR, 128), jnp.float32),
            jax.ShapeDtypeStruct((NR, 128), jnp.float32),
        ],
    )(degp4)


def _q1_body(x_ref, n_ref, w_ref, o_ref):
    q = jnp.dot(x_ref[...], w_ref[...], preferred_element_type=jnp.float32)
    o_ref[...] = q * n_ref[...]


def _tc_q1(xp8, no16, w1p8):
    return pl.pallas_call(
        _q1_body,
        grid=(49,),
        in_specs=[
            pl.BlockSpec((256, 256), lambda i: (i, 0)),
            pl.BlockSpec((256, 128), lambda i: (i, 0)),
            pl.BlockSpec((256, 128), lambda i: (0, 0)),
        ],
        out_specs=pl.BlockSpec((256, 128), lambda i: (i, 0)),
        out_shape=jax.ShapeDtypeStruct((N8, 128), jnp.float32),
    )(xp8, no16, w1p8)


def _mid_body(agg_ref, ni_ref, no_ref, b_ref, w_ref, o_ref):
    agg = agg_ref[...]
    h = jax.nn.relu((agg[0] + agg[1]) * ni_ref[...] + b_ref[...])
    q = jnp.dot(h, w_ref[...], preferred_element_type=jnp.float32)
    o_ref[...] = q * no_ref[...]


def _tc_mid(aggp1r, ni16r, no8, b1t, w2p16):
    return pl.pallas_call(
        _mid_body,
        grid=(49,),
        in_specs=[
            pl.BlockSpec((NC, 128, 256), lambda i: (0, i, 0)),
            pl.BlockSpec((128, 256), lambda i: (i, 0)),
            pl.BlockSpec((128, 128), lambda i: (i, 0)),
            pl.BlockSpec((1, 256), lambda i: (0, 0)),
            pl.BlockSpec((256, 128), lambda i: (0, 0)),
        ],
        out_specs=pl.BlockSpec((128, 128), lambda i: (i, 0)),
        out_shape=jax.ShapeDtypeStruct((N16, 128), jnp.float32),
    )(aggp1r, ni16r, no8, b1t, w2p16)


def _out_body(agg_ref, ni_ref, b_ref, o_ref):
    agg = agg_ref[...]
    o_ref[...] = (agg[0] + agg[1]) * ni_ref[...] + b_ref[...]


def _tc_out(aggp2r, ni8, b2t):
    return pl.pallas_call(
        _out_body,
        grid=(49,),
        in_specs=[
            pl.BlockSpec((NC, 128, 128), lambda i: (0, i, 0)),
            pl.BlockSpec((128, 128), lambda i: (i, 0)),
            pl.BlockSpec((1, 128), lambda i: (0, 0)),
        ],
        out_specs=pl.BlockSpec((128, 128), lambda i: (i, 0)),
        out_shape=jax.ShapeDtypeStruct((N16, 128), jnp.float32),
    )(aggp2r, ni8, b2t)


def _lane_repeat(nvec, k, rows):
    return jnp.repeat(nvec.reshape(-1), k).reshape(rows, 128)


@jax.jit
def kernel(x, edge_index, W1, b1, W2, b2):
    pad = jnp.full((EPAD - E,), N, dtype=jnp.int32)
    srcr = jnp.concatenate([edge_index[0], pad]).reshape(ER, 128)
    dstr = jnp.concatenate([edge_index[1], pad]).reshape(ER, 128)
    xp8 = jnp.pad(x, ((0, NPAD - N), (0, 0))).reshape(N8, 256)
    w1p8 = jnp.kron(jnp.eye(8, dtype=jnp.float32), W1)
    w2p16 = jnp.kron(jnp.eye(16, dtype=jnp.float32), W2)
    b1t = jnp.tile(b1, 16).reshape(1, 256)
    b2t = jnp.tile(b2, 16).reshape(1, 128)

    degp = _make_deg_kernel()(
        srcr, dstr, jnp.zeros((ZCH,), jnp.float32), jnp.ones((128,), jnp.float32)
    )
    no, ni = _tc_norms(degp.reshape(NC, 2, NR, 128))
    no16 = _lane_repeat(no, 16, N8)
    ni16r = _lane_repeat(ni, 16, N8).reshape(N16, 256)
    no8 = _lane_repeat(no, 8, N16)
    ni8 = _lane_repeat(ni, 8, N16)

    q1 = _tc_q1(xp8, no16, w1p8).reshape(NPAD, D_HID)
    aggp1 = _make_edge_kernel(D_HID)(
        srcr, dstr, q1, jnp.zeros((ZCH, D_HID), jnp.float32)
    )
    q2 = _tc_mid(
        aggp1.reshape(NC, N16, 256), ni16r, no8, b1t, w2p16
    ).reshape(NPAD, D_OUT)
    aggp2 = _make_edge_kernel(D_OUT)(
        srcr, dstr, q2, jnp.zeros((ZCH, D_OUT), jnp.float32)
    )
    out = _tc_out(aggp2.reshape(NC, N16, 128), ni8, b2t)
    return out.reshape(NPAD, D_OUT)[:N]


# speedup vs baseline: 46.2069x; 1.0485x over previous
import functools

import jax
import jax.numpy as jnp
from jax import lax
from jax.experimental import pallas as pl
from jax.experimental.pallas import tpu as pltpu
from jax.experimental.pallas import tpu_sc as plsc

N = 100000
E = 1600000
D_IN = 32
D_HID = 16
D_OUT = 8

NC = 2
NS = 16
NPAD = 100352
EPAD = 1605632
ER = EPAD // 128
ROWS_W = ER // (NC * NS)
KIN = 14
GOUT = ROWS_W // KIN
TPS = NPAD // NS
NZC = 14
ZCH = TPS // NZC

@functools.cache
def _mesh():
    return plsc.VectorSubcoreMesh(
        core_axis_name="c", subcore_axis_name="s", num_cores=NC, num_subcores=NS
    )


def _worker_base(c, s):
    return (c * NS + s) * ROWS_W


@functools.cache
def _make_deg_kernel():
    return functools.partial(
        pl.kernel,
        out_type=jax.ShapeDtypeStruct((NC * 2 * NPAD,), jnp.float32),
        mesh=_mesh(),
        compiler_params=pltpu.CompilerParams(use_tc_tiling_on_sc=False),
        scratch_types=[
            pltpu.VMEM((2, KIN, 128), jnp.int32),
            pltpu.VMEM((2, KIN, 128), jnp.int32),
            pltpu.VMEM((128,), jnp.float32),
            pltpu.VMEM((ZCH,), jnp.float32),
            pltpu.VMEM_SHARED((NPAD,), jnp.float32),
            pltpu.VMEM_SHARED((NPAD,), jnp.float32),
            pltpu.SemaphoreType.DMA,
            pltpu.SemaphoreType.DMA,
        ],
    )(_deg_body)


def _deg_body(srcr, dstr, zer, one, out, src_v, dst_v, ones_v, zb, dego, degi, ss0, ss1):
    ssems = (ss0, ss1)
    c = lax.axis_index("c")
    s = lax.axis_index("s")
    pltpu.sync_copy(zer, zb)
    pltpu.sync_copy(one, ones_v)
    for i in range(NZC):
        off = s * TPS + i * ZCH
        pltpu.sync_copy(zb, dego.at[pl.ds(off, ZCH)])
        pltpu.sync_copy(zb, degi.at[pl.ds(off, ZCH)])
    plsc.subcore_barrier()
    w = _worker_base(c, s)

    def stage(g, p):
        base = w + g * KIN
        pltpu.sync_copy(srcr.at[pl.ds(base, KIN)], src_v.at[p])
        pltpu.sync_copy(dstr.at[pl.ds(base, KIN)], dst_v.at[p])

    def fire(p):
        for j in range(KIN):
            pltpu.async_copy(ones_v, dego.at[src_v.at[p, j]], ssems[p], add=True)
            pltpu.async_copy(ones_v, degi.at[dst_v.at[p, j]], ssems[p], add=True)

    def drain(p):
        for j in range(KIN):
            pltpu.make_async_copy(ones_v, dego.at[src_v.at[p, j]], ssems[p]).wait()
            pltpu.make_async_copy(ones_v, degi.at[dst_v.at[p, j]], ssems[p]).wait()

    stage(0, 0)
    stage(1, 1)
    fire(0)

    def body(i, carry):
        g = i * 2

        @pl.when(g + 1 < GOUT)
        def _():
            fire(1)

        drain(0)

        @pl.when(g + 2 < GOUT)
        def _():
            stage(g + 2, 0)
            fire(0)

        @pl.when(g + 1 < GOUT)
        def _():
            drain(1)

        @pl.when(g + 3 < GOUT)
        def _():
            stage(g + 3, 1)

        return carry

    lax.fori_loop(0, (GOUT + 1) // 2, body, 0)
    plsc.subcore_barrier()
    for i in range(NZC):
        off = s * TPS + i * ZCH
        pltpu.sync_copy(
            dego.at[pl.ds(off, ZCH)], out.at[pl.ds(c * 2 * NPAD + off, ZCH)]
        )
        pltpu.sync_copy(
            degi.at[pl.ds(off, ZCH)], out.at[pl.ds((c * 2 + 1) * NPAD + off, ZCH)]
        )


@functools.cache
def _make_edge_kernel(d):
    kin = 4 if d == 16 else 14
    gout = ROWS_W // kin

    @functools.partial(
        pl.kernel,
        out_type=jax.ShapeDtypeStruct((NC, NPAD, d), jnp.float32),
        mesh=_mesh(),
        compiler_params=pltpu.CompilerParams(use_tc_tiling_on_sc=False),
        scratch_types=[
            pltpu.VMEM((2, kin, 128), jnp.int32),
            pltpu.VMEM((2, kin, 128), jnp.int32),
            pltpu.VMEM((2, kin * 128, d), jnp.float32),
            pltpu.VMEM_SHARED((NPAD, d), jnp.float32),
            pltpu.SemaphoreType.DMA,
            pltpu.SemaphoreType.DMA,
            pltpu.SemaphoreType.DMA,
            pltpu.SemaphoreType.DMA,
        ],
    )
    def _edge_kernel(
        srcr, dstr, q, zer, out, src_v, dst_v, rows_v, agg, gs0, gs1, ss0, ss1
    ):
        gsems = (gs0, gs1)
        ssems = (ss0, ss1)
        c = lax.axis_index("c")
        s = lax.axis_index("s")
        pltpu.sync_copy(zer, rows_v.at[0, pl.ds(0, ZCH)])
        for i in range(NZC):
            off = s * TPS + i * ZCH
            pltpu.sync_copy(rows_v.at[0, pl.ds(0, ZCH)], agg.at[pl.ds(off, ZCH)])
        plsc.subcore_barrier()
        w = _worker_base(c, s)

        def stage(g, p):
            base = w + g * kin
            pltpu.sync_copy(srcr.at[pl.ds(base, kin)], src_v.at[p])
            pltpu.sync_copy(dstr.at[pl.ds(base, kin)], dst_v.at[p])

        def fire_gathers(p):
            for j in range(kin):
                pltpu.async_copy(
                    q.at[src_v.at[p, j]],
                    rows_v.at[p, pl.ds(j * 128, 128)],
                    gsems[p],
                )

        def drain_gathers(p):
            for j in range(kin):
                pltpu.make_async_copy(
                    q.at[src_v.at[p, j]],
                    rows_v.at[p, pl.ds(j * 128, 128)],
                    gsems[p],
                ).wait()

        def fire_scatters(p):
            for j in range(kin):
                pltpu.async_copy(
                    rows_v.at[p, pl.ds(j * 128, 128)],
                    agg.at[dst_v.at[p, j]],
                    ssems[p],
                    add=True,
                )

        def drain_scatters(p):
            for j in range(kin):
                pltpu.make_async_copy(
                    rows_v.at[p, pl.ds(j * 128, 128)],
                    agg.at[dst_v.at[p, j]],
                    ssems[p],
                ).wait()

        stage(0, 0)
        stage(1, 1)
        fire_gathers(0)

        def half(g, p):
            @pl.when(g + 1 < gout)
            def _():
                fire_gathers(1 - p)

            drain_gathers(p)
            fire_scatters(p)

            @pl.when(g + 2 < gout)
            def _():
                stage(g + 2, p)

        def body(i, carry):
            g = i * 2

            @pl.when(g > 0)
            def _():
                drain_scatters(1)

            half(g, 0)
            drain_scatters(0)
            half(g + 1, 1)
            return carry

        lax.fori_loop(0, gout // 2, body, 0)
        if gout % 2 == 1:
            drain_scatters(1)
            drain_gathers(0)
            fire_scatters(0)
            drain_scatters(0)
        else:
            drain_scatters(1)
        plsc.subcore_barrier()
        for i in range(NZC):
            off = s * TPS + i * ZCH
            pltpu.sync_copy(agg.at[pl.ds(off, ZCH)], out.at[c, pl.ds(off, ZCH)])

    return _edge_kernel

NR = NPAD // 128
N8 = NPAD // 8
N16 = NPAD // 16


def _norm_body(deg_ref, no_ref, ni_ref):
    deg = deg_ref[...]
    no_ref[...] = lax.rsqrt(jnp.maximum(deg[0, 0] + deg[1, 0], 1.0))
    ni_ref[...] = lax.rsqrt(jnp.maximum(deg[0, 1] + deg[1, 1], 1.0))


def _tc_norms(degp4):
    return pl.pallas_call(
        _norm_body,
        out_shape=[
            jax.ShapeDtypeStruct((NR, 128), jnp.float32),
            jax.ShapeDtypeStruct((NR, 128), jnp.float32),
        ],
    )(degp4)


def _q1_body(x_ref, n_ref, w_ref, o_ref):
    q = jnp.dot(x_ref[...], w_ref[...], preferred_element_type=jnp.float32)
    o_ref[...] = q * n_ref[...]


def _tc_q1(xp8, no16, w1p8):
    return pl.pallas_call(
        _q1_body,
        grid=(49,),
        in_specs=[
            pl.BlockSpec((256, 256), lambda i: (i, 0)),
            pl.BlockSpec((256, 128), lambda i: (i, 0)),
            pl.BlockSpec((256, 128), lambda i: (0, 0)),
        ],
        out_specs=pl.BlockSpec((256, 128), lambda i: (i, 0)),
        out_shape=jax.ShapeDtypeStruct((N8, 128), jnp.float32),
    )(xp8, no16, w1p8)


def _mid_body(agg_ref, ni_ref, no_ref, b_ref, w_ref, o_ref):
    agg = agg_ref[...]
    h = jax.nn.relu((agg[0] + agg[1]) * ni_ref[...] + b_ref[...])
    q = jnp.dot(h, w_ref[...], preferred_element_type=jnp.float32)
    o_ref[...] = q * no_ref[...]


def _tc_mid(aggp1r, ni16r, no8, b1t, w2p16):
    return pl.pallas_call(
        _mid_body,
        grid=(49,),
        in_specs=[
            pl.BlockSpec((NC, 128, 256), lambda i: (0, i, 0)),
            pl.BlockSpec((128, 256), lambda i: (i, 0)),
            pl.BlockSpec((128, 128), lambda i: (i, 0)),
            pl.BlockSpec((1, 256), lambda i: (0, 0)),
            pl.BlockSpec((256, 128), lambda i: (0, 0)),
        ],
        out_specs=pl.BlockSpec((128, 128), lambda i: (i, 0)),
        out_shape=jax.ShapeDtypeStruct((N16, 128), jnp.float32),
    )(aggp1r, ni16r, no8, b1t, w2p16)


def _out_body(agg_ref, ni_ref, b_ref, o_ref):
    agg = agg_ref[...]
    o_ref[...] = (agg[0] + agg[1]) * ni_ref[...] + b_ref[...]


def _tc_out(aggp2r, ni8, b2t):
    return pl.pallas_call(
        _out_body,
        grid=(49,),
        in_specs=[
            pl.BlockSpec((NC, 128, 128), lambda i: (0, i, 0)),
            pl.BlockSpec((128, 128), lambda i: (i, 0)),
            pl.BlockSpec((1, 128), lambda i: (0, 0)),
        ],
        out_specs=pl.BlockSpec((128, 128), lambda i: (i, 0)),
        out_shape=jax.ShapeDtypeStruct((N // 16, 128), jnp.float32),
    )(aggp2r, ni8, b2t)


def _lane_repeat(nvec, k, rows):
    return jnp.repeat(nvec.reshape(-1), k).reshape(rows, 128)


@jax.jit
def kernel(x, edge_index, W1, b1, W2, b2):
    pad = jnp.full((EPAD - E,), N, dtype=jnp.int32)
    srcr = jnp.concatenate([edge_index[0], pad]).reshape(ER, 128)
    dstr = jnp.concatenate([edge_index[1], pad]).reshape(ER, 128)
    xp8 = jnp.pad(x, ((0, NPAD - N), (0, 0))).reshape(N8, 256)
    w1p8 = jnp.kron(jnp.eye(8, dtype=jnp.float32), W1)
    w2p16 = jnp.kron(jnp.eye(16, dtype=jnp.float32), W2)
    b1t = jnp.tile(b1, 16).reshape(1, 256)
    b2t = jnp.tile(b2, 16).reshape(1, 128)

    degp = _make_deg_kernel()(
        srcr, dstr, jnp.zeros((ZCH,), jnp.float32), jnp.ones((128,), jnp.float32)
    )
    no, ni = _tc_norms(degp.reshape(NC, 2, NR, 128))
    no16 = _lane_repeat(no, 16, N8)
    ni16r = _lane_repeat(ni, 16, N8).reshape(N16, 256)
    no8 = _lane_repeat(no, 8, N16)
    ni8 = _lane_repeat(ni, 8, N16)

    q1 = _tc_q1(xp8, no16, w1p8).reshape(NPAD, D_HID)
    aggp1 = _make_edge_kernel(D_HID)(
        srcr, dstr, q1, jnp.zeros((ZCH, D_HID), jnp.float32)
    )
    q2 = _tc_mid(
        aggp1.reshape(NC, N16, 256), ni16r, no8, b1t, w2p16
    ).reshape(NPAD, D_OUT)
    aggp2 = _make_edge_kernel(D_OUT)(
        srcr, dstr, q2, jnp.zeros((ZCH, D_OUT), jnp.float32)
    )
    out = _tc_out(aggp2.reshape(NC, N16, 128), ni8, b2t)
    return out.reshape(N, D_OUT)


# speedup vs baseline: 46.8495x; 1.0139x over previous
import functools

import jax
import jax.numpy as jnp
from jax import lax
from jax.experimental import pallas as pl
from jax.experimental.pallas import tpu as pltpu
from jax.experimental.pallas import tpu_sc as plsc

N = 100000
E = 1600000
D_IN = 32
D_HID = 16
D_OUT = 8

NC = 2
NS = 16
NPAD = 100352
EPAD = 1605632
ER = EPAD // 128
ROWS_W = ER // (NC * NS)
KIN = 14
GOUT = ROWS_W // KIN
TPS = NPAD // NS
NZC = 14
ZCH = TPS // NZC

@functools.cache
def _mesh():
    return plsc.VectorSubcoreMesh(
        core_axis_name="c", subcore_axis_name="s", num_cores=NC, num_subcores=NS
    )


def _worker_base(c, s):
    return (c * NS + s) * ROWS_W


@functools.cache
def _make_deg_kernel():
    return functools.partial(
        pl.kernel,
        out_type=jax.ShapeDtypeStruct((NC * 2 * NPAD,), jnp.float32),
        mesh=_mesh(),
        compiler_params=pltpu.CompilerParams(use_tc_tiling_on_sc=False),
        scratch_types=[
            pltpu.VMEM((2, KIN, 128), jnp.int32),
            pltpu.VMEM((2, KIN, 128), jnp.int32),
            pltpu.VMEM((128,), jnp.float32),
            pltpu.VMEM((ZCH,), jnp.float32),
            pltpu.VMEM_SHARED((NPAD,), jnp.float32),
            pltpu.VMEM_SHARED((NPAD,), jnp.float32),
            pltpu.SemaphoreType.DMA,
            pltpu.SemaphoreType.DMA,
        ],
    )(_deg_body)


def _deg_body(srcr, dstr, zer, one, out, src_v, dst_v, ones_v, zb, dego, degi, ss0, ss1):
    ssems = (ss0, ss1)
    c = lax.axis_index("c")
    s = lax.axis_index("s")
    pltpu.sync_copy(zer, zb)
    pltpu.sync_copy(one, ones_v)
    for i in range(NZC):
        off = s * TPS + i * ZCH
        pltpu.sync_copy(zb, dego.at[pl.ds(off, ZCH)])
        pltpu.sync_copy(zb, degi.at[pl.ds(off, ZCH)])
    plsc.subcore_barrier()
    w = _worker_base(c, s)

    def stage(g, p):
        base = w + g * KIN
        pltpu.sync_copy(srcr.at[pl.ds(base, KIN)], src_v.at[p])
        pltpu.sync_copy(dstr.at[pl.ds(base, KIN)], dst_v.at[p])

    def fire(p):
        for j in range(KIN):
            pltpu.async_copy(ones_v, dego.at[src_v.at[p, j]], ssems[p], add=True)
            pltpu.async_copy(ones_v, degi.at[dst_v.at[p, j]], ssems[p], add=True)

    def drain(p):
        for j in range(KIN):
            pltpu.make_async_copy(ones_v, dego.at[src_v.at[p, j]], ssems[p]).wait()
            pltpu.make_async_copy(ones_v, degi.at[dst_v.at[p, j]], ssems[p]).wait()

    stage(0, 0)
    stage(1, 1)
    fire(0)

    def body(i, carry):
        g = i * 2

        @pl.when(g + 1 < GOUT)
        def _():
            fire(1)

        drain(0)

        @pl.when(g + 2 < GOUT)
        def _():
            stage(g + 2, 0)
            fire(0)

        @pl.when(g + 1 < GOUT)
        def _():
            drain(1)

        @pl.when(g + 3 < GOUT)
        def _():
            stage(g + 3, 1)

        return carry

    lax.fori_loop(0, (GOUT + 1) // 2, body, 0)
    plsc.subcore_barrier()
    for i in range(NZC):
        off = s * TPS + i * ZCH
        pltpu.sync_copy(
            dego.at[pl.ds(off, ZCH)], out.at[pl.ds(c * 2 * NPAD + off, ZCH)]
        )
        pltpu.sync_copy(
            degi.at[pl.ds(off, ZCH)], out.at[pl.ds((c * 2 + 1) * NPAD + off, ZCH)]
        )


@functools.cache
def _make_edge_kernel(d):
    kin = 4 if d == 16 else 14
    gout = ROWS_W // kin

    @functools.partial(
        pl.kernel,
        out_type=jax.ShapeDtypeStruct((NC, NPAD, d), jnp.float32),
        mesh=_mesh(),
        compiler_params=pltpu.CompilerParams(use_tc_tiling_on_sc=False),
        scratch_types=[
            pltpu.VMEM((2, kin, 128), jnp.int32),
            pltpu.VMEM((2, kin, 128), jnp.int32),
            pltpu.VMEM((2, kin * 128, d), jnp.float32),
            pltpu.VMEM_SHARED((NPAD, d), jnp.float32),
            pltpu.SemaphoreType.DMA,
            pltpu.SemaphoreType.DMA,
            pltpu.SemaphoreType.DMA,
            pltpu.SemaphoreType.DMA,
        ],
    )
    def _edge_kernel(
        srcr, dstr, q, zer, out, src_v, dst_v, rows_v, agg, gs0, gs1, ss0, ss1
    ):
        gsems = (gs0, gs1)
        ssems = (ss0, ss1)
        c = lax.axis_index("c")
        s = lax.axis_index("s")
        pltpu.sync_copy(zer, rows_v.at[0, pl.ds(0, ZCH)])
        for i in range(NZC):
            off = s * TPS + i * ZCH
            pltpu.sync_copy(rows_v.at[0, pl.ds(0, ZCH)], agg.at[pl.ds(off, ZCH)])
        plsc.subcore_barrier()
        w = _worker_base(c, s)

        def stage(g, p):
            base = w + g * kin
            pltpu.sync_copy(srcr.at[pl.ds(base, kin)], src_v.at[p])
            pltpu.sync_copy(dstr.at[pl.ds(base, kin)], dst_v.at[p])

        def fire_gathers(p):
            for j in range(kin):
                pltpu.async_copy(
                    q.at[src_v.at[p, j]],
                    rows_v.at[p, pl.ds(j * 128, 128)],
                    gsems[p],
                )

        def drain_gathers(p):
            for j in range(kin):
                pltpu.make_async_copy(
                    q.at[src_v.at[p, j]],
                    rows_v.at[p, pl.ds(j * 128, 128)],
                    gsems[p],
                ).wait()

        def fire_scatters(p):
            for j in range(kin):
                pltpu.async_copy(
                    rows_v.at[p, pl.ds(j * 128, 128)],
                    agg.at[dst_v.at[p, j]],
                    ssems[p],
                    add=True,
                )

        def drain_scatters(p):
            for j in range(kin):
                pltpu.make_async_copy(
                    rows_v.at[p, pl.ds(j * 128, 128)],
                    agg.at[dst_v.at[p, j]],
                    ssems[p],
                ).wait()

        stage(0, 0)
        stage(1, 1)
        fire_gathers(0)

        def half(g, p):
            @pl.when(g + 1 < gout)
            def _():
                fire_gathers(1 - p)

            drain_gathers(p)
            fire_scatters(p)

            @pl.when(g + 2 < gout)
            def _():
                stage(g + 2, p)

        def body(i, carry):
            g = i * 2

            @pl.when(g > 0)
            def _():
                drain_scatters(1)

            half(g, 0)
            drain_scatters(0)
            half(g + 1, 1)
            return carry

        lax.fori_loop(0, gout // 2, body, 0)
        if gout % 2 == 1:
            drain_scatters(1)
            drain_gathers(0)
            fire_scatters(0)
            drain_scatters(0)
        else:
            drain_scatters(1)
        plsc.subcore_barrier()
        for i in range(NZC):
            off = s * TPS + i * ZCH
            pltpu.sync_copy(agg.at[pl.ds(off, ZCH)], out.at[c, pl.ds(off, ZCH)])

    return _edge_kernel

NR = NPAD // 128
N8 = NPAD // 8
N16 = NPAD // 16


def _norm_body(deg_ref, no_ref, ni_ref):
    deg = deg_ref[...]
    no_ref[...] = lax.rsqrt(jnp.maximum(deg[0, 0] + deg[1, 0], 1.0))
    ni_ref[...] = lax.rsqrt(jnp.maximum(deg[0, 1] + deg[1, 1], 1.0))


def _tc_norms(degp4):
    return pl.pallas_call(
        _norm_body,
        out_shape=[
            jax.ShapeDtypeStruct((NR, 128), jnp.float32),
            jax.ShapeDtypeStruct((NR, 128), jnp.float32),
        ],
    )(degp4)


def _p1_body(x_ref, w_ref, o_ref):
    o_ref[...] = jnp.dot(x_ref[...], w_ref[...], preferred_element_type=jnp.float32)


def _tc_p1(xp8, w1p8):
    return pl.pallas_call(
        _p1_body,
        grid=(49,),
        in_specs=[
            pl.BlockSpec((256, 256), lambda i: (i, 0)),
            pl.BlockSpec((256, 128), lambda i: (0, 0)),
        ],
        out_specs=pl.BlockSpec((256, 128), lambda i: (i, 0)),
        out_shape=jax.ShapeDtypeStruct((N8, 128), jnp.float32),
    )(xp8, w1p8)


def _scale_body(p_ref, n_ref, o_ref):
    o_ref[...] = p_ref[...] * n_ref[...]


def _tc_scale(p1, no16):
    return pl.pallas_call(
        _scale_body,
        grid=(49,),
        in_specs=[
            pl.BlockSpec((256, 128), lambda i: (i, 0)),
            pl.BlockSpec((256, 128), lambda i: (i, 0)),
        ],
        out_specs=pl.BlockSpec((256, 128), lambda i: (i, 0)),
        out_shape=jax.ShapeDtypeStruct((N8, 128), jnp.float32),
    )(p1, no16)


def _mid_body(agg_ref, ni_ref, no_ref, b_ref, w_ref, o_ref):
    agg = agg_ref[...]
    h = jax.nn.relu((agg[0] + agg[1]) * ni_ref[...] + b_ref[...])
    q = jnp.dot(h, w_ref[...], preferred_element_type=jnp.float32)
    o_ref[...] = q * no_ref[...]


def _tc_mid(aggp1r, ni16r, no8, b1t, w2p16):
    return pl.pallas_call(
        _mid_body,
        grid=(49,),
        in_specs=[
            pl.BlockSpec((NC, 128, 256), lambda i: (0, i, 0)),
            pl.BlockSpec((128, 256), lambda i: (i, 0)),
            pl.BlockSpec((128, 128), lambda i: (i, 0)),
            pl.BlockSpec((1, 256), lambda i: (0, 0)),
            pl.BlockSpec((256, 128), lambda i: (0, 0)),
        ],
        out_specs=pl.BlockSpec((128, 128), lambda i: (i, 0)),
        out_shape=jax.ShapeDtypeStruct((N16, 128), jnp.float32),
    )(aggp1r, ni16r, no8, b1t, w2p16)


def _out_body(agg_ref, ni_ref, b_ref, o_ref):
    agg = agg_ref[...]
    o_ref[...] = (agg[0] + agg[1]) * ni_ref[...] + b_ref[...]


def _tc_out(aggp2r, ni8, b2t):
    return pl.pallas_call(
        _out_body,
        grid=(49,),
        in_specs=[
            pl.BlockSpec((NC, 128, 128), lambda i: (0, i, 0)),
            pl.BlockSpec((128, 128), lambda i: (i, 0)),
            pl.BlockSpec((1, 128), lambda i: (0, 0)),
        ],
        out_specs=pl.BlockSpec((128, 128), lambda i: (i, 0)),
        out_shape=jax.ShapeDtypeStruct((N // 16, 128), jnp.float32),
    )(aggp2r, ni8, b2t)


def _lane_repeat(nvec, k, rows):
    return jnp.repeat(nvec.reshape(-1), k).reshape(rows, 128)


@jax.jit
def kernel(x, edge_index, W1, b1, W2, b2):
    pad = jnp.full((EPAD - E,), N, dtype=jnp.int32)
    srcr = jnp.concatenate([edge_index[0], pad]).reshape(ER, 128)
    dstr = jnp.concatenate([edge_index[1], pad]).reshape(ER, 128)
    xp8 = jnp.pad(x, ((0, NPAD - N), (0, 0))).reshape(N8, 256)
    w1p8 = jnp.kron(jnp.eye(8, dtype=jnp.float32), W1)
    w2p16 = jnp.kron(jnp.eye(16, dtype=jnp.float32), W2)
    b1t = jnp.tile(b1, 16).reshape(1, 256)
    b2t = jnp.tile(b2, 16).reshape(1, 128)

    p1 = _tc_p1(xp8, w1p8)
    degp = _make_deg_kernel()(
        srcr, dstr, jnp.zeros((ZCH,), jnp.float32), jnp.ones((128,), jnp.float32)
    )
    no, ni = _tc_norms(degp.reshape(NC, 2, NR, 128))
    no16 = _lane_repeat(no, 16, N8)
    ni16r = _lane_repeat(ni, 16, N8).reshape(N16, 256)
    no8 = _lane_repeat(no, 8, N16)
    ni8 = _lane_repeat(ni, 8, N16)

    q1 = _tc_scale(p1, no16).reshape(NPAD, D_HID)
    aggp1 = _make_edge_kernel(D_HID)(
        srcr, dstr, q1, jnp.zeros((ZCH, D_HID), jnp.float32)
    )
    q2 = _tc_mid(
        aggp1.reshape(NC, N16, 256), ni16r, no8, b1t, w2p16
    ).reshape(NPAD, D_OUT)
    aggp2 = _make_edge_kernel(D_OUT)(
        srcr, dstr, q2, jnp.zeros((ZCH, D_OUT), jnp.float32)
    )
    out = _tc_out(aggp2.reshape(NC, N16, 128), ni8, b2t)
    return out.reshape(N, D_OUT)
